# baseline (speedup 1.0000x reference)
import numpy as np
import jax
import jax.numpy as jnp
from jax import lax
from jax.experimental import pallas as pl
from jax.experimental.pallas import tpu as pltpu

_FEAT_STRIDE = 16
_PRE = 12000
_POST = 2000
_THR = 0.7
_S = 128
_NB = 94
_NPAD = _S * _NB


def _whctrs(a):
    w = a[2] - a[0] + 1.0
    h = a[3] - a[1] + 1.0
    return w, h, a[0] + 0.5 * (w - 1), a[1] + 0.5 * (h - 1)


def _mkanchors(ws, hs, xc, yc):
    ws = np.asarray(ws, dtype=np.float64).reshape(-1, 1)
    hs = np.asarray(hs, dtype=np.float64).reshape(-1, 1)
    return np.hstack((xc - 0.5 * (ws - 1), yc - 0.5 * (hs - 1),
                      xc + 0.5 * (ws - 1), yc + 0.5 * (hs - 1)))


def _gen_anchors():
    ratios = np.array([0.5, 1.0, 2.0])
    scales = np.array([8, 16, 32])
    base = np.array([1.0, 1.0, 16.0, 16.0]) - 1
    w, h, xc, yc = _whctrs(base)
    size = w * h
    ws = np.round(np.sqrt(size / ratios))
    hs = np.round(ws * ratios)
    ra = _mkanchors(ws, hs, xc, yc)
    rows = []
    for i in range(ra.shape[0]):
        w, h, xc, yc = _whctrs(ra[i])
        rows.append(_mkanchors(w * scales, h * scales, xc, yc))
    return np.vstack(rows).astype(np.float32)


def _anchor_grid(H, W):
    a0 = _gen_anchors()
    A = a0.shape[0]
    shift_x = np.arange(W) * _FEAT_STRIDE
    shift_y = np.arange(H) * _FEAT_STRIDE
    sx, sy = np.meshgrid(shift_x, shift_y)
    shifts = np.vstack((sx.ravel(), sy.ravel(), sx.ravel(), sy.ravel()))
    shifts = shifts.transpose().astype(np.float32)
    K = shifts.shape[0]
    anchors = a0[None, :, :] + shifts[:, None, :]
    return anchors.reshape(1, K * A, 4), A


_CH = 512
_KCAP = 2304
_WIN = 256


def _nms_body(x1_ref, y1_ref, x2_ref, y2_ref, col_ref, keptm_ref,
              k0_ref, k1_ref, k2_ref, k3_ref, kc_ref):
    f32 = jnp.float32
    i32 = jnp.int32
    tri = (lax.broadcasted_iota(i32, (_S, _S), 0)
           < lax.broadcasted_iota(i32, (_S, _S), 1))
    ut = (lax.broadcasted_iota(i32, (_S, _S), 0)
          <= lax.broadcasted_iota(i32, (_S, _S), 1)).astype(f32)
    ones_ch = jnp.ones((1, _CH), f32)
    iota_ch = lax.broadcasted_iota(i32, (_CH, 1), 0)
    iota_win = lax.broadcasted_iota(i32, (_WIN, 1), 0)
    lane = lax.broadcasted_iota(i32, (1, _S), 1)

    kc_ref[0] = 0
    keptm_ref[0, :, :] = jnp.zeros((_NB, _S), f32)
    k0_ref[:, :] = jnp.zeros((_KCAP, 4), f32)
    k1_ref[:, :] = jnp.zeros((_KCAP, 4), f32)
    k2_ref[:, :] = jnp.zeros((_KCAP, 4), f32)
    k3_ref[:, :] = jnp.zeros((_KCAP, 4), f32)

    def block_body(b, carry):
        @pl.when(kc_ref[0] < _POST)
        def _():
            kc = kc_ref[0]
            cx1 = x1_ref[0, pl.ds(b, 1), :]
            cy1 = y1_ref[0, pl.ds(b, 1), :]
            cx2 = x2_ref[0, pl.ds(b, 1), :]
            cy2 = y2_ref[0, pl.ds(b, 1), :]
            carea = (cx2 - cx1 + 1.0) * (cy2 - cy1 + 1.0)

            def chunk_body(c, scnt):
                q0 = k0_ref[pl.ds(c * _CH, _CH), :].astype(i32)
                q1 = k1_ref[pl.ds(c * _CH, _CH), :].astype(i32)
                q2 = k2_ref[pl.ds(c * _CH, _CH), :].astype(i32)
                q3 = k3_ref[pl.ds(c * _CH, _CH), :].astype(i32)
                kk = lax.bitcast_convert_type(
                    ((q3 * 256 + q2) * 256 + q1) * 256 + q0, f32)
                kx1 = kk[:, 0:1]
                ky1 = kk[:, 1:2]
                kx2 = kk[:, 2:3]
                ky2 = kk[:, 3:4]
                karea = (kx2 - kx1 + 1.0) * (ky2 - ky1 + 1.0)
                kvalid = (c * _CH + iota_ch) < kc
                a1 = jnp.maximum(kx1, cx1)
                b1 = jnp.maximum(ky1, cy1)
                a2 = jnp.minimum(kx2, cx2)
                b2 = jnp.minimum(ky2, cy2)
                ww = jnp.maximum(0.0, a2 - a1 + 1.0)
                hh = jnp.maximum(0.0, b2 - b1 + 1.0)
                it = ww * hh
                ov = it / (karea + carea - it)
                supm = ((ov > _THR) & kvalid).astype(f32)
                cnt = lax.dot_general(ones_ch, supm, (((1,), (0,)), ((), ())),
                                      preferred_element_type=f32)
                return scnt + cnt

            nch = (kc + _CH - 1) // _CH
            scnt = lax.fori_loop(0, nch, chunk_body, jnp.zeros((1, _S), f32))
            alive = ((b * _S + lane) < _PRE).astype(f32) * (scnt < 0.5)

            kx1 = col_ref[0, pl.ds(b * _S, _S), 0:1]
            ky1 = col_ref[0, pl.ds(b * _S, _S), 1:2]
            kx2 = col_ref[0, pl.ds(b * _S, _S), 2:3]
            ky2 = col_ref[0, pl.ds(b * _S, _S), 3:4]
            karea = (kx2 - kx1 + 1.0) * (ky2 - ky1 + 1.0)
            xx1 = jnp.maximum(kx1, cx1)
            yy1 = jnp.maximum(ky1, cy1)
            xx2 = jnp.minimum(kx2, cx2)
            yy2 = jnp.minimum(ky2, cy2)
            w = jnp.maximum(0.0, xx2 - xx1 + 1.0)
            h = jnp.maximum(0.0, yy2 - yy1 + 1.0)
            inter = w * h
            ovr = inter / (karea + carea - inter)
            mtri = ((ovr > _THR) & tri).astype(f32)

            def fix_cond(st):
                return st[1]

            def fix_body(st):
                keep, _ = st
                sup = lax.dot_general(keep, mtri, (((1,), (0,)), ((), ())),
                                      preferred_element_type=f32)
                nk = alive * (sup < 0.5).astype(f32)
                return nk, jnp.any(nk != keep)

            keep, _ = lax.while_loop(fix_cond, fix_body,
                                     (alive, jnp.bool_(True)))

            keptm_ref[0, pl.ds(b, 1), :] = keep
            nk = jnp.sum(keep).astype(i32)

            @pl.when(nk > 0)
            def _():
                pos = (kc - 1) + lax.dot_general(
                    keep, ut, (((1,), (0,)), ((), ())),
                    preferred_element_type=f32).astype(i32)
                w0 = (kc // _S) * _S
                oh = ((iota_win + w0 == pos) & (keep > 0.5)).astype(f32)
                cbits = lax.bitcast_convert_type(
                    jnp.concatenate([cx1, cy1, cx2, cy2], axis=0), i32)
                c0 = (cbits % 256).astype(f32)
                r1 = cbits // 256
                c1 = (r1 % 256).astype(f32)
                r2 = r1 // 256
                c2 = (r2 % 256).astype(f32)
                c3 = (r2 // 256).astype(f32)
                for cc, kref in ((c0, k0_ref), (c1, k1_ref),
                                 (c2, k2_ref), (c3, k3_ref)):
                    u = lax.dot_general(oh, cc, (((1,), (1,)), ((), ())),
                                        preferred_element_type=f32)
                    kref[pl.ds(w0, _WIN), :] = kref[pl.ds(w0, _WIN), :] + u
                kc_ref[0] = kc + nk

        return carry

    lax.fori_loop(0, _NB, block_body, 0)


def _run_nms(x1r, y1r, x2r, y2r, colp):
    B = x1r.shape[0]
    spec_row = pl.BlockSpec((1, _NB, _S), lambda i: (i, 0, 0))
    spec_col = pl.BlockSpec((1, _NPAD, 4), lambda i: (i, 0, 0))
    return pl.pallas_call(
        _nms_body,
        grid=(B,),
        in_specs=[spec_row, spec_row, spec_row, spec_row, spec_col],
        out_specs=spec_row,
        out_shape=jax.ShapeDtypeStruct((B, _NB, _S), jnp.float32),
        scratch_shapes=[pltpu.VMEM((_KCAP, 4), jnp.float32),
                        pltpu.VMEM((_KCAP, 4), jnp.float32),
                        pltpu.VMEM((_KCAP, 4), jnp.float32),
                        pltpu.VMEM((_KCAP, 4), jnp.float32),
                        pltpu.SMEM((1,), jnp.int32)],
    )(x1r, y1r, x2r, y2r, colp)


_NR = 162


def _xform_body(dx_ref, dy_ref, dw_ref, dh_ref, aw_ref, ah_ref, acx_ref,
                acy_ref, wm1_ref, hm1_ref, x1_ref, y1_ref, x2_ref, y2_ref):
    aw = aw_ref[:, :]
    ah = ah_ref[:, :]
    pcx = dx_ref[0] * aw + acx_ref[:, :]
    pcy = dy_ref[0] * ah + acy_ref[:, :]
    pw = jnp.exp(dw_ref[0]) * aw
    ph = jnp.exp(dh_ref[0]) * ah
    wm1 = wm1_ref[0]
    hm1 = hm1_ref[0]
    x1_ref[0] = jnp.minimum(jnp.maximum(pcx - 0.5 * pw, 0.0), wm1)
    y1_ref[0] = jnp.minimum(jnp.maximum(pcy - 0.5 * ph, 0.0), hm1)
    x2_ref[0] = jnp.minimum(jnp.maximum(pcx + 0.5 * pw, 0.0), wm1)
    y2_ref[0] = jnp.minimum(jnp.maximum(pcy + 0.5 * ph, 0.0), hm1)


def _run_xform(dx, dy, dw, dh, aw, ah, acx, acy, wm1, hm1):
    B = dx.shape[0]
    spec_b = pl.BlockSpec((1, _NR, _S), lambda i: (i, 0, 0))
    spec_c = pl.BlockSpec((_NR, _S), lambda i: (0, 0))
    spec_s = pl.BlockSpec((1, 1, _S), lambda i: (i, 0, 0))
    out = jax.ShapeDtypeStruct((B, _NR, _S), jnp.float32)
    return pl.pallas_call(
        _xform_body,
        grid=(B,),
        in_specs=[spec_b, spec_b, spec_b, spec_b,
                  spec_c, spec_c, spec_c, spec_c, spec_s, spec_s],
        out_specs=[spec_b, spec_b, spec_b, spec_b],
        out_shape=[out, out, out, out],
    )(dx, dy, dw, dh, aw, ah, acx, acy, wm1, hm1)


def kernel(scores, bbox_deltas, im_info):
    B = scores.shape[0]
    H, W = scores.shape[2], scores.shape[3]
    anchors_np, A = _anchor_grid(H, W)
    anchors = jnp.asarray(anchors_np)
    anchors = jnp.broadcast_to(anchors, (B, anchors.shape[1], 4))

    sc = scores[:, A:, :, :]
    sc_flat = jnp.transpose(sc, (0, 2, 3, 1)).reshape(B, -1)
    deltas = jnp.transpose(bbox_deltas, (0, 2, 3, 1)).reshape(B, -1, 4)

    a_np = anchors_np[0]
    aw_np = a_np[:, 2] - a_np[:, 0] + np.float32(1.0)
    ah_np = a_np[:, 3] - a_np[:, 1] + np.float32(1.0)
    acx_np = a_np[:, 0] + np.float32(0.5) * aw_np
    acy_np = a_np[:, 1] + np.float32(0.5) * ah_np
    aw = jnp.asarray(aw_np.reshape(_NR, _S))
    ah = jnp.asarray(ah_np.reshape(_NR, _S))
    acx = jnp.asarray(acx_np.reshape(_NR, _S))
    acy = jnp.asarray(acy_np.reshape(_NR, _S))
    dx = deltas[:, :, 0].reshape(B, _NR, _S)
    dy = deltas[:, :, 1].reshape(B, _NR, _S)
    dw = deltas[:, :, 2].reshape(B, _NR, _S)
    dh = deltas[:, :, 3].reshape(B, _NR, _S)
    wm1 = jnp.broadcast_to((im_info[:, 1:2] - 1.0)[:, :, None], (B, 1, _S))
    hm1 = jnp.broadcast_to((im_info[:, 0:1] - 1.0)[:, :, None], (B, 1, _S))
    x1, y1, x2, y2 = _run_xform(dx, dy, dw, dh, aw, ah, acx, acy, wm1, hm1)
    proposals = jnp.stack([x1.reshape(B, -1), y1.reshape(B, -1),
                           x2.reshape(B, -1), y2.reshape(B, -1)], axis=2)

    order = jnp.argsort(-sc_flat, axis=1)[:, :_PRE]
    props = jnp.take_along_axis(proposals, order[:, :, None], axis=1)
    propsp = jnp.pad(props, ((0, 0), (0, _NPAD - _PRE), (0, 0)))

    x1r = propsp[:, :, 0].reshape(B, _NB, _S)
    y1r = propsp[:, :, 1].reshape(B, _NB, _S)
    x2r = propsp[:, :, 2].reshape(B, _NB, _S)
    y2r = propsp[:, :, 3].reshape(B, _NB, _S)

    keptm = _run_nms(x1r, y1r, x2r, y2r, propsp)

    flat = keptm.reshape(B, _NPAD) > 0.5
    pos = jnp.cumsum(flat.astype(jnp.int32), axis=1) - 1
    posc = jnp.where(flat & (pos < _POST), pos, _POST)
    out4 = jax.vmap(
        lambda p, bx: jnp.zeros((_POST + 1, 4), jnp.float32).at[p].add(bx)
    )(posc, propsp)
    out4 = out4[:, :_POST]
    col0 = jnp.broadcast_to(
        jnp.arange(B, dtype=jnp.float32)[:, None, None], (B, _POST, 1))
    return jnp.concatenate([col0, out4], axis=2)


# speedup vs baseline: 3.9928x; 3.9928x over previous
import numpy as np
import jax
import jax.numpy as jnp
from jax import lax
from jax.experimental import pallas as pl
from jax.experimental.pallas import tpu as pltpu

_FEAT_STRIDE = 16
_PRE = 12000
_POST = 2000
_THR = 0.7
_S = 128
_NB = 94
_NPAD = _S * _NB


def _whctrs(a):
    w = a[2] - a[0] + 1.0
    h = a[3] - a[1] + 1.0
    return w, h, a[0] + 0.5 * (w - 1), a[1] + 0.5 * (h - 1)


def _mkanchors(ws, hs, xc, yc):
    ws = np.asarray(ws, dtype=np.float64).reshape(-1, 1)
    hs = np.asarray(hs, dtype=np.float64).reshape(-1, 1)
    return np.hstack((xc - 0.5 * (ws - 1), yc - 0.5 * (hs - 1),
                      xc + 0.5 * (ws - 1), yc + 0.5 * (hs - 1)))


def _gen_anchors():
    ratios = np.array([0.5, 1.0, 2.0])
    scales = np.array([8, 16, 32])
    base = np.array([1.0, 1.0, 16.0, 16.0]) - 1
    w, h, xc, yc = _whctrs(base)
    size = w * h
    ws = np.round(np.sqrt(size / ratios))
    hs = np.round(ws * ratios)
    ra = _mkanchors(ws, hs, xc, yc)
    rows = []
    for i in range(ra.shape[0]):
        w, h, xc, yc = _whctrs(ra[i])
        rows.append(_mkanchors(w * scales, h * scales, xc, yc))
    return np.vstack(rows).astype(np.float32)


def _anchor_grid(H, W):
    a0 = _gen_anchors()
    A = a0.shape[0]
    shift_x = np.arange(W) * _FEAT_STRIDE
    shift_y = np.arange(H) * _FEAT_STRIDE
    sx, sy = np.meshgrid(shift_x, shift_y)
    shifts = np.vstack((sx.ravel(), sy.ravel(), sx.ravel(), sy.ravel()))
    shifts = shifts.transpose().astype(np.float32)
    K = shifts.shape[0]
    anchors = a0[None, :, :] + shifts[:, None, :]
    return anchors.reshape(1, K * A, 4), A


_CH = 512
_KCAP = 2304
_WIN = 256


def _nms_body(x1_ref, y1_ref, x2_ref, y2_ref, col_ref, keptm_ref,
              k0_ref, k1_ref, k2_ref, k3_ref, kc_ref):
    f32 = jnp.float32
    i32 = jnp.int32
    tri = (lax.broadcasted_iota(i32, (_S, _S), 0)
           < lax.broadcasted_iota(i32, (_S, _S), 1))
    ut = (lax.broadcasted_iota(i32, (_S, _S), 0)
          <= lax.broadcasted_iota(i32, (_S, _S), 1)).astype(f32)
    ones_ch = jnp.ones((1, _CH), f32)
    iota_ch = lax.broadcasted_iota(i32, (_CH, 1), 0)
    iota_win = lax.broadcasted_iota(i32, (_WIN, 1), 0)
    lane = lax.broadcasted_iota(i32, (1, _S), 1)

    kc_ref[0] = 0
    keptm_ref[0, :, :] = jnp.zeros((_NB, _S), f32)
    k0_ref[:, :] = jnp.zeros((_KCAP, 4), f32)
    k1_ref[:, :] = jnp.zeros((_KCAP, 4), f32)
    k2_ref[:, :] = jnp.zeros((_KCAP, 4), f32)
    k3_ref[:, :] = jnp.zeros((_KCAP, 4), f32)

    def block_body(b, carry):
        @pl.when(kc_ref[0] < _POST)
        def _():
            kc = kc_ref[0]
            cx1 = x1_ref[0, pl.ds(b, 1), :]
            cy1 = y1_ref[0, pl.ds(b, 1), :]
            cx2 = x2_ref[0, pl.ds(b, 1), :]
            cy2 = y2_ref[0, pl.ds(b, 1), :]
            carea = (cx2 - cx1 + 1.0) * (cy2 - cy1 + 1.0)

            def chunk_body(c, scnt):
                q0 = k0_ref[pl.ds(c * _CH, _CH), :].astype(i32)
                q1 = k1_ref[pl.ds(c * _CH, _CH), :].astype(i32)
                q2 = k2_ref[pl.ds(c * _CH, _CH), :].astype(i32)
                q3 = k3_ref[pl.ds(c * _CH, _CH), :].astype(i32)
                kk = lax.bitcast_convert_type(
                    ((q3 * 256 + q2) * 256 + q1) * 256 + q0, f32)
                kx1 = kk[:, 0:1]
                ky1 = kk[:, 1:2]
                kx2 = kk[:, 2:3]
                ky2 = kk[:, 3:4]
                karea = (kx2 - kx1 + 1.0) * (ky2 - ky1 + 1.0)
                kvalid = (c * _CH + iota_ch) < kc
                a1 = jnp.maximum(kx1, cx1)
                b1 = jnp.maximum(ky1, cy1)
                a2 = jnp.minimum(kx2, cx2)
                b2 = jnp.minimum(ky2, cy2)
                ww = jnp.maximum(0.0, a2 - a1 + 1.0)
                hh = jnp.maximum(0.0, b2 - b1 + 1.0)
                it = ww * hh
                ov = it / (karea + carea - it)
                supm = ((ov > _THR) & kvalid).astype(f32)
                cnt = lax.dot_general(ones_ch, supm, (((1,), (0,)), ((), ())),
                                      preferred_element_type=f32)
                return scnt + cnt

            nch = (kc + _CH - 1) // _CH
            scnt = lax.fori_loop(0, nch, chunk_body, jnp.zeros((1, _S), f32))
            alive = ((b * _S + lane) < _PRE).astype(f32) * (scnt < 0.5)

            kx1 = col_ref[0, pl.ds(b * _S, _S), 0:1]
            ky1 = col_ref[0, pl.ds(b * _S, _S), 1:2]
            kx2 = col_ref[0, pl.ds(b * _S, _S), 2:3]
            ky2 = col_ref[0, pl.ds(b * _S, _S), 3:4]
            karea = (kx2 - kx1 + 1.0) * (ky2 - ky1 + 1.0)
            xx1 = jnp.maximum(kx1, cx1)
            yy1 = jnp.maximum(ky1, cy1)
            xx2 = jnp.minimum(kx2, cx2)
            yy2 = jnp.minimum(ky2, cy2)
            w = jnp.maximum(0.0, xx2 - xx1 + 1.0)
            h = jnp.maximum(0.0, yy2 - yy1 + 1.0)
            inter = w * h
            ovr = inter / (karea + carea - inter)
            mtri = ((ovr > _THR) & tri).astype(f32)

            def fix_cond(st):
                return st[1]

            def fix_body(st):
                keep, _ = st
                sup = lax.dot_general(keep, mtri, (((1,), (0,)), ((), ())),
                                      preferred_element_type=f32)
                nk = alive * (sup < 0.5).astype(f32)
                return nk, jnp.any(nk != keep)

            keep, _ = lax.while_loop(fix_cond, fix_body,
                                     (alive, jnp.bool_(True)))

            keptm_ref[0, pl.ds(b, 1), :] = keep
            nk = jnp.sum(keep).astype(i32)

            @pl.when(nk > 0)
            def _():
                pos = (kc - 1) + lax.dot_general(
                    keep, ut, (((1,), (0,)), ((), ())),
                    preferred_element_type=f32).astype(i32)
                w0 = (kc // _S) * _S
                oh = ((iota_win + w0 == pos) & (keep > 0.5)).astype(f32)
                cbits = lax.bitcast_convert_type(
                    jnp.concatenate([cx1, cy1, cx2, cy2], axis=0), i32)
                c0 = (cbits % 256).astype(f32)
                r1 = cbits // 256
                c1 = (r1 % 256).astype(f32)
                r2 = r1 // 256
                c2 = (r2 % 256).astype(f32)
                c3 = (r2 // 256).astype(f32)
                for cc, kref in ((c0, k0_ref), (c1, k1_ref),
                                 (c2, k2_ref), (c3, k3_ref)):
                    u = lax.dot_general(oh, cc, (((1,), (1,)), ((), ())),
                                        preferred_element_type=f32)
                    kref[pl.ds(w0, _WIN), :] = kref[pl.ds(w0, _WIN), :] + u
                kc_ref[0] = kc + nk

        return carry

    lax.fori_loop(0, _NB, block_body, 0)


def _run_nms(x1r, y1r, x2r, y2r, colp):
    B = x1r.shape[0]
    spec_row = pl.BlockSpec((1, _NB, _S), lambda i: (i, 0, 0))
    spec_col = pl.BlockSpec((1, _NPAD, 4), lambda i: (i, 0, 0))
    return pl.pallas_call(
        _nms_body,
        grid=(B,),
        in_specs=[spec_row, spec_row, spec_row, spec_row, spec_col],
        out_specs=spec_row,
        out_shape=jax.ShapeDtypeStruct((B, _NB, _S), jnp.float32),
        scratch_shapes=[pltpu.VMEM((_KCAP, 4), jnp.float32),
                        pltpu.VMEM((_KCAP, 4), jnp.float32),
                        pltpu.VMEM((_KCAP, 4), jnp.float32),
                        pltpu.VMEM((_KCAP, 4), jnp.float32),
                        pltpu.SMEM((1,), jnp.int32)],
    )(x1r, y1r, x2r, y2r, colp)


_NR = 162


def _xform_body(dx_ref, dy_ref, dw_ref, dh_ref, aw_ref, ah_ref, acx_ref,
                acy_ref, wm1_ref, hm1_ref, x1_ref, y1_ref, x2_ref, y2_ref):
    aw = aw_ref[:, :]
    ah = ah_ref[:, :]
    pcx = dx_ref[0] * aw + acx_ref[:, :]
    pcy = dy_ref[0] * ah + acy_ref[:, :]
    pw = jnp.exp(dw_ref[0]) * aw
    ph = jnp.exp(dh_ref[0]) * ah
    wm1 = wm1_ref[0]
    hm1 = hm1_ref[0]
    x1_ref[0] = jnp.minimum(jnp.maximum(pcx - 0.5 * pw, 0.0), wm1)
    y1_ref[0] = jnp.minimum(jnp.maximum(pcy - 0.5 * ph, 0.0), hm1)
    x2_ref[0] = jnp.minimum(jnp.maximum(pcx + 0.5 * pw, 0.0), wm1)
    y2_ref[0] = jnp.minimum(jnp.maximum(pcy + 0.5 * ph, 0.0), hm1)


def _run_xform(dx, dy, dw, dh, aw, ah, acx, acy, wm1, hm1):
    B = dx.shape[0]
    spec_b = pl.BlockSpec((1, _NR, _S), lambda i: (i, 0, 0))
    spec_c = pl.BlockSpec((_NR, _S), lambda i: (0, 0))
    spec_s = pl.BlockSpec((1, 1, _S), lambda i: (i, 0, 0))
    out = jax.ShapeDtypeStruct((B, _NR, _S), jnp.float32)
    return pl.pallas_call(
        _xform_body,
        grid=(B,),
        in_specs=[spec_b, spec_b, spec_b, spec_b,
                  spec_c, spec_c, spec_c, spec_c, spec_s, spec_s],
        out_specs=[spec_b, spec_b, spec_b, spec_b],
        out_shape=[out, out, out, out],
    )(dx, dy, dw, dh, aw, ah, acx, acy, wm1, hm1)


def kernel(scores, bbox_deltas, im_info):
    B = scores.shape[0]
    H, W = scores.shape[2], scores.shape[3]
    anchors_np, A = _anchor_grid(H, W)
    anchors = jnp.asarray(anchors_np)
    anchors = jnp.broadcast_to(anchors, (B, anchors.shape[1], 4))

    sc = scores[:, A:, :, :]
    sc_flat = jnp.transpose(sc, (0, 2, 3, 1)).reshape(B, -1)
    deltas = jnp.transpose(bbox_deltas, (0, 2, 3, 1)).reshape(B, -1, 4)

    a_np = anchors_np[0]
    aw_np = a_np[:, 2] - a_np[:, 0] + np.float32(1.0)
    ah_np = a_np[:, 3] - a_np[:, 1] + np.float32(1.0)
    acx_np = a_np[:, 0] + np.float32(0.5) * aw_np
    acy_np = a_np[:, 1] + np.float32(0.5) * ah_np
    aw = jnp.asarray(aw_np.reshape(_NR, _S))
    ah = jnp.asarray(ah_np.reshape(_NR, _S))
    acx = jnp.asarray(acx_np.reshape(_NR, _S))
    acy = jnp.asarray(acy_np.reshape(_NR, _S))
    dx = deltas[:, :, 0].reshape(B, _NR, _S)
    dy = deltas[:, :, 1].reshape(B, _NR, _S)
    dw = deltas[:, :, 2].reshape(B, _NR, _S)
    dh = deltas[:, :, 3].reshape(B, _NR, _S)
    wm1 = jnp.broadcast_to((im_info[:, 1:2] - 1.0)[:, :, None], (B, 1, _S))
    hm1 = jnp.broadcast_to((im_info[:, 0:1] - 1.0)[:, :, None], (B, 1, _S))
    x1, y1, x2, y2 = _run_xform(dx, dy, dw, dh, aw, ah, acx, acy, wm1, hm1)

    order = jnp.argsort(-sc_flat, axis=1)[:, :_PRE]
    order_p = jnp.pad(order, ((0, 0), (0, _NPAD - _PRE)))
    sx1 = jnp.take_along_axis(x1.reshape(B, -1), order_p, axis=1)
    sy1 = jnp.take_along_axis(y1.reshape(B, -1), order_p, axis=1)
    sx2 = jnp.take_along_axis(x2.reshape(B, -1), order_p, axis=1)
    sy2 = jnp.take_along_axis(y2.reshape(B, -1), order_p, axis=1)
    propsp = jnp.stack([sx1, sy1, sx2, sy2], axis=2)

    x1r = sx1.reshape(B, _NB, _S)
    y1r = sy1.reshape(B, _NB, _S)
    x2r = sx2.reshape(B, _NB, _S)
    y2r = sy2.reshape(B, _NB, _S)

    keptm = _run_nms(x1r, y1r, x2r, y2r, propsp)

    flat = keptm.reshape(B, _NPAD) > 0.5
    pos = jnp.cumsum(flat.astype(jnp.int32), axis=1) - 1
    posc = jnp.where(flat & (pos < _POST), pos, _POST)
    out4 = jax.vmap(
        lambda p, bx: jnp.zeros((_POST + 1, 4), jnp.float32).at[p].add(bx)
    )(posc, propsp)
    out4 = out4[:, :_POST]
    col0 = jnp.broadcast_to(
        jnp.arange(B, dtype=jnp.float32)[:, None, None], (B, _POST, 1))
    return jnp.concatenate([col0, out4], axis=2)
